# Initial kernel scaffold; baseline (speedup 1.0000x reference)
import jax
import jax.numpy as jnp
from jax.experimental import pallas as pl
from jax.experimental.pallas import tpu as pltpu

E = 8
N_GROUP = 4
TOPK_GROUP = 2
TOP_K = 2
GROUP_SIZE = E // N_GROUP
ROUTED_SCALING = 2.5
NEG = -1e30


def _logits_kernel(x_ref, gw_ref, out_ref):
    out_ref[...] = jax.lax.dot_general(
        gw_ref[...], x_ref[...], (((1,), (1,)), ((), ())),
        preferred_element_type=jnp.float32,
        precision=jax.lax.Precision.HIGHEST)


def _routing_kernel(lg_ref, bias_ref, w_ref):
    sc = [1.0 / (1.0 + jnp.exp(-lg_ref[e])) for e in range(E)]
    sb = [sc[e] + bias_ref[e] for e in range(E)]
    g = [sb[GROUP_SIZE * gi] + sb[GROUP_SIZE * gi + 1] for gi in range(N_GROUP)]
    gmask = []
    for gi in range(N_GROUP):
        r = jnp.zeros_like(g[gi])
        for gj in range(N_GROUP):
            if gj == gi:
                continue
            beats = (g[gj] > g[gi]) if gj > gi else (g[gj] >= g[gi])
            r = r + beats.astype(jnp.float32)
        gmask.append(r < TOPK_GROUP)
    ms = [jnp.where(gmask[e // GROUP_SIZE], sb[e], NEG) for e in range(E)]
    wts = []
    for ei in range(E):
        r = jnp.zeros_like(ms[ei])
        for ej in range(E):
            if ej == ei:
                continue
            beats = (ms[ej] > ms[ei]) if ej > ei else (ms[ej] >= ms[ei])
            r = r + beats.astype(jnp.float32)
        sel = r < TOP_K
        wts.append(jnp.where(sel, sc[ei], 0.0))
    denom = wts[0]
    for e in range(1, E):
        denom = denom + wts[e]
    denom = denom + 1e-20
    for e in range(E):
        w_ref[e] = wts[e] / denom


def _moe_kernel(x_ref, wt_ref, w_up_ref, w_down_ref, sup_ref, sdown_ref,
                out_ref):
    x = x_ref[...]
    xb = x.astype(jnp.bfloat16)
    bt, d = x.shape
    acc = jnp.zeros((bt, d), jnp.float32)
    for e in range(E):
        h = jax.lax.dot_general(
            xb, w_up_ref[e], (((1,), (1,)), ((), ())),
            preferred_element_type=jnp.float32)
        h = jnp.square(jnp.maximum(h, 0.0)).astype(jnp.bfloat16)
        y = jax.lax.dot_general(
            h, w_down_ref[e], (((1,), (1,)), ((), ())),
            preferred_element_type=jnp.float32)
        acc = acc + wt_ref[:, e:e + 1] * y
    hs = jax.lax.dot_general(
        xb, sup_ref[...], (((1,), (1,)), ((), ())),
        preferred_element_type=jnp.float32)
    hs = jnp.square(jnp.maximum(hs, 0.0)).astype(jnp.bfloat16)
    ys = jax.lax.dot_general(
        hs, sdown_ref[...], (((1,), (1,)), ((), ())),
        preferred_element_type=jnp.float32)
    out_ref[...] = acc * ROUTED_SCALING + ys


def kernel(hidden_states, gate_w, e_score_correction_bias, w_up, w_down,
           shared_w_up, shared_w_down):
    t, d = hidden_states.shape
    e, d_ff, _ = w_up.shape
    d_ff_sh = shared_w_up.shape[0]
    assert e == E

    logits_t = pl.pallas_call(
        _logits_kernel,
        out_shape=jax.ShapeDtypeStruct((E, t), jnp.float32),
    )(hidden_states, gate_w)

    tb = t // 128
    lg3 = logits_t.reshape(E, tb, 128)
    bias3 = e_score_correction_bias.reshape(E, 1, 1)
    w3 = pl.pallas_call(
        _routing_kernel,
        out_shape=jax.ShapeDtypeStruct((E, tb, 128), jnp.float32),
    )(lg3, bias3)
    wt = w3.reshape(E, t).T

    bt = 256
    grid = (t // bt,)
    w_up_b = w_up.astype(jnp.bfloat16)
    w_down_b = w_down.astype(jnp.bfloat16)
    sup_b = shared_w_up.astype(jnp.bfloat16)
    sdown_b = shared_w_down.astype(jnp.bfloat16)

    out = pl.pallas_call(
        _moe_kernel,
        grid=grid,
        in_specs=[
            pl.BlockSpec((bt, d), lambda i: (i, 0)),
            pl.BlockSpec((bt, E), lambda i: (i, 0)),
            pl.BlockSpec((E, d_ff, d), lambda i: (0, 0, 0)),
            pl.BlockSpec((E, d, d_ff), lambda i: (0, 0, 0)),
            pl.BlockSpec((d_ff_sh, d), lambda i: (0, 0)),
            pl.BlockSpec((d, d_ff_sh), lambda i: (0, 0)),
        ],
        out_specs=pl.BlockSpec((bt, d), lambda i: (i, 0)),
        out_shape=jax.ShapeDtypeStruct((t, d), jnp.float32),
        compiler_params=pltpu.CompilerParams(
            dimension_semantics=("arbitrary",),
            vmem_limit_bytes=100 * 1024 * 1024,
        ),
    )(hidden_states, wt, w_up_b, w_down_b, sup_b, sdown_b)
    return out


# speedup vs baseline: 1.6581x; 1.6581x over previous
import jax
import jax.numpy as jnp
from jax.experimental import pallas as pl
from jax.experimental.pallas import tpu as pltpu

E = 8
N_GROUP = 4
TOPK_GROUP = 2
TOP_K = 2
GROUP_SIZE = E // N_GROUP
ROUTED_SCALING = 2.5
NEG = -1e30


def _logits_kernel(x_ref, gw_ref, out_ref):
    out_ref[...] = jax.lax.dot_general(
        gw_ref[...], x_ref[...], (((1,), (1,)), ((), ())),
        preferred_element_type=jnp.float32)


def _routing_kernel(lg_ref, bias_ref, w_ref):
    sc = [1.0 / (1.0 + jnp.exp(-lg_ref[e])) for e in range(E)]
    sb = [sc[e] + bias_ref[e] for e in range(E)]
    g = [sb[GROUP_SIZE * gi] + sb[GROUP_SIZE * gi + 1] for gi in range(N_GROUP)]
    gmask = []
    for gi in range(N_GROUP):
        r = jnp.zeros_like(g[gi])
        for gj in range(N_GROUP):
            if gj == gi:
                continue
            beats = (g[gj] > g[gi]) if gj > gi else (g[gj] >= g[gi])
            r = r + beats.astype(jnp.float32)
        gmask.append(r < TOPK_GROUP)
    ms = [jnp.where(gmask[e // GROUP_SIZE], sb[e], NEG) for e in range(E)]
    wts = []
    for ei in range(E):
        r = jnp.zeros_like(ms[ei])
        for ej in range(E):
            if ej == ei:
                continue
            beats = (ms[ej] > ms[ei]) if ej > ei else (ms[ej] >= ms[ei])
            r = r + beats.astype(jnp.float32)
        sel = r < TOP_K
        wts.append(jnp.where(sel, sc[ei], 0.0))
    denom = wts[0]
    for e in range(1, E):
        denom = denom + wts[e]
    denom = denom + 1e-20
    for e in range(E):
        w_ref[e] = wts[e] / denom


def _moe_kernel(x_ref, wt_ref, w_up_ref, w_down_ref, sup_ref, sdown_ref,
                out_ref):
    x = x_ref[...]
    xb = x.astype(jnp.bfloat16)
    bt, d = x.shape
    acc = jnp.zeros((bt, d), jnp.float32)
    for e in range(E):
        h = jax.lax.dot_general(
            xb, w_up_ref[e], (((1,), (1,)), ((), ())),
            preferred_element_type=jnp.float32)
        h = jnp.square(jnp.maximum(h, 0.0)).astype(jnp.bfloat16)
        y = jax.lax.dot_general(
            h, w_down_ref[e], (((1,), (1,)), ((), ())),
            preferred_element_type=jnp.float32)
        acc = acc + wt_ref[:, e:e + 1] * y
    hs = jax.lax.dot_general(
        xb, sup_ref[...], (((1,), (1,)), ((), ())),
        preferred_element_type=jnp.float32)
    hs = jnp.square(jnp.maximum(hs, 0.0)).astype(jnp.bfloat16)
    ys = jax.lax.dot_general(
        hs, sdown_ref[...], (((1,), (1,)), ((), ())),
        preferred_element_type=jnp.float32)
    out_ref[...] = acc * ROUTED_SCALING + ys


def kernel(hidden_states, gate_w, e_score_correction_bias, w_up, w_down,
           shared_w_up, shared_w_down):
    t, d = hidden_states.shape
    e, d_ff, _ = w_up.shape
    d_ff_sh = shared_w_up.shape[0]
    assert e == E

    logits_t = pl.pallas_call(
        _logits_kernel,
        out_shape=jax.ShapeDtypeStruct((E, t), jnp.float32),
    )(hidden_states, gate_w)

    tb = t // 128
    lg3 = logits_t.reshape(E, tb, 128)
    bias3 = e_score_correction_bias.reshape(E, 1, 1)
    w3 = pl.pallas_call(
        _routing_kernel,
        out_shape=jax.ShapeDtypeStruct((E, tb, 128), jnp.float32),
    )(lg3, bias3)
    wt = w3.reshape(E, t).T

    bt = 256
    grid = (t // bt,)
    w_up_b = w_up.astype(jnp.bfloat16)
    w_down_b = w_down.astype(jnp.bfloat16)
    sup_b = shared_w_up.astype(jnp.bfloat16)
    sdown_b = shared_w_down.astype(jnp.bfloat16)

    out = pl.pallas_call(
        _moe_kernel,
        grid=grid,
        in_specs=[
            pl.BlockSpec((bt, d), lambda i: (i, 0)),
            pl.BlockSpec((bt, E), lambda i: (i, 0)),
            pl.BlockSpec((E, d_ff, d), lambda i: (0, 0, 0)),
            pl.BlockSpec((E, d, d_ff), lambda i: (0, 0, 0)),
            pl.BlockSpec((d_ff_sh, d), lambda i: (0, 0)),
            pl.BlockSpec((d, d_ff_sh), lambda i: (0, 0)),
        ],
        out_specs=pl.BlockSpec((bt, d), lambda i: (i, 0)),
        out_shape=jax.ShapeDtypeStruct((t, d), jnp.float32),
        compiler_params=pltpu.CompilerParams(
            dimension_semantics=("arbitrary",),
            vmem_limit_bytes=100 * 1024 * 1024,
        ),
    )(hidden_states, wt, w_up_b, w_down_b, sup_b, sdown_b)
    return out
